# baseline (speedup 1.0000x reference)
import functools

import jax
import jax.numpy as jnp
from jax import lax
from jax.experimental import pallas as pl
from jax.experimental.pallas import tpu as pltpu
from jax.experimental.pallas import tpu_sc as plsc

LINPUT = 256
COUNT = 10
BUCKET_SIZE = 6
BATCH = 16384

NC = 2
NS = 16
L = 16
NW = NC * NS
BPW = BATCH // NW
XCH = 128
NCHUNK = BPW // XCH
NJ = LINPUT // L

_GDN = lax.GatherDimensionNumbers(
    offset_dims=(), collapsed_slice_dims=(0,), start_index_map=(0,))


def _permute(v, idx):
    return lax.gather(v, idx[:, None], _GDN, (1,),
                      mode=lax.GatherScatterMode.PROMISE_IN_BOUNDS)


def _sc_body(x_hbm, ply_hbm, w_hbm, b_hbm, out_hbm,
             w_v, b_v, ply_v, xbuf, out_v, sem0, sem1):
    wid = lax.axis_index("s") * NC + lax.axis_index("c")
    base = wid * BPW
    sems = (sem0, sem1)

    def xcopy(ch, buf):
        return pltpu.make_async_copy(
            x_hbm.at[pl.ds(base + ch * XCH, XCH)], xbuf.at[buf], sems[buf])

    xcopy(0, 0).start()
    pltpu.sync_copy(w_hbm, w_v)
    pltpu.sync_copy(b_hbm, b_v)
    pltpu.sync_copy(ply_hbm.at[pl.ds(base, BPW)], ply_v)

    lane = lax.iota(jnp.int32, L)
    bvec = b_v[pl.ds(0, L)]
    bs = [bvec[c0] for c0 in range(COUNT)]

    for ch in range(NCHUNK):
        buf = ch & 1
        if ch + 1 < NCHUNK:
            xcopy(ch + 1, 1 - buf).start()
        xcopy(ch, buf).wait()

        def group_body(g, carry, ch=ch, buf=buf):
            gs = ch * XCH + g * L
            cvec = lax.shift_right_logical(ply_v[pl.ds(gs, L)] * 10923, 16)
            stack = []
            for s in range(L):
                row = g * L + s
                woff = cvec[s] * LINPUT
                parts = [xbuf[buf, row, pl.ds(k * L, L)] * w_v[pl.ds(woff + k * L, L)]
                         for k in range(4)]
                for jc in range(4, NJ):
                    k = jc & 3
                    parts[k] = parts[k] + (xbuf[buf, row, pl.ds(jc * L, L)]
                                           * w_v[pl.ds(woff + jc * L, L)])
                node = (0, (parts[0] + parts[1]) + (parts[2] + parts[3]))
                while stack and stack[-1][0] == node[0]:
                    lvl, a = stack.pop()
                    m = 1 << lvl
                    sel = (lane & m) == 0
                    perm = lane ^ m
                    c = node[1]
                    node = (lvl + 1,
                            jnp.where(sel, a + _permute(a, perm),
                                      c + _permute(c, perm)))
                stack.append(node)
            dots = stack[0][1]
            badd = jnp.zeros((L,), jnp.float32)
            for c0 in range(COUNT):
                badd = jnp.where(cvec == c0, bs[c0], badd)
            out_v[pl.ds(gs, L)] = dots + badd
            return carry

        lax.fori_loop(0, XCH // L, group_body, 0)

    pltpu.sync_copy(out_v, out_hbm.at[pl.ds(base, BPW)])


@jax.jit
def _run(x_pa, ply, wf, bf):
    mesh = plsc.VectorSubcoreMesh(core_axis_name="c", subcore_axis_name="s")
    f = functools.partial(
        pl.kernel,
        out_type=jax.ShapeDtypeStruct((BATCH,), jnp.float32),
        mesh=mesh,
        scratch_types=[
            pltpu.VMEM((COUNT * LINPUT,), jnp.float32),
            pltpu.VMEM((L,), jnp.float32),
            pltpu.VMEM((BPW,), jnp.int32),
            pltpu.VMEM((2, XCH, LINPUT), jnp.float32),
            pltpu.VMEM((BPW,), jnp.float32),
            pltpu.SemaphoreType.DMA,
            pltpu.SemaphoreType.DMA,
        ],
    )(_sc_body)
    return f(x_pa, ply, wf, bf)


def kernel(x_pa, ply, W, b):
    wf = W.reshape(COUNT * LINPUT)
    bf = jnp.zeros((L,), jnp.float32).at[:COUNT].set(b.reshape(COUNT))
    out = _run(x_pa, ply, wf, bf)
    return out.reshape(BATCH, 1)
